# Initial kernel scaffold; baseline (speedup 1.0000x reference)
import functools

import numpy as np
import jax
import jax.numpy as jnp
from jax import lax
from jax.experimental import pallas as pl
from jax.experimental.pallas import tpu as pltpu
from jax.experimental.pallas import tpu_sc as plsc

B = 4096
F = 26
V = 100000
D = 32
DIN = 13
NF = F + 1

NC = 2
NS = 16
NW = NC * NS
CHUNK = B * F // NW
NIDX = 128
NCH = CHUNK // NIDX

_OFF = ((np.arange(CHUNK, dtype=np.int64) % F) * V).astype(np.int32)
_OFF2D = _OFF.reshape(NCH, NIDX)

_sc_mesh = plsc.VectorSubcoreMesh(core_axis_name="c", subcore_axis_name="s")


@functools.partial(
    pl.kernel,
    mesh=_sc_mesh,
    out_type=jax.ShapeDtypeStruct((B * F, D), jnp.float32),
    scratch_types=[
        pltpu.VMEM((NCH, NIDX), jnp.int32),
        pltpu.VMEM((NCH, NIDX), jnp.int32),
        pltpu.VMEM((CHUNK, D), jnp.float32),
        pltpu.SemaphoreType.DMA,
    ],
)
def _sc_gather(idx_hbm, off_hbm, table_hbm, out_hbm, idx_v, off_v, rows_v, sem):
    wid = lax.axis_index("s") * NC + lax.axis_index("c")
    row0 = wid * NCH
    pltpu.sync_copy(idx_hbm.at[pl.ds(row0, NCH)], idx_v)
    pltpu.sync_copy(off_hbm, off_v)
    for j in range(NCH):
        for i in range(NIDX // 16):
            sl = pl.ds(i * 16, 16)
            idx_v[j, sl] = idx_v[j, sl] + off_v[j, sl]
    cps = []
    for j in range(NCH):
        cps.append(
            pltpu.async_copy(
                table_hbm.at[idx_v.at[j]],
                rows_v.at[pl.ds(j * NIDX, NIDX)],
                sem,
            )
        )
    for cp in cps:
        cp.wait()
    pltpu.sync_copy(rows_v, out_hbm.at[pl.ds(wid * CHUNK, CHUNK)])


_TRI_I, _TRI_J = np.tril_indices(NF, -1)
_PAIRPOS = {(int(i), int(j)): c for c, (i, j) in enumerate(zip(_TRI_I, _TRI_J))}

_SLAB_BASE = [0] * (F + 1)
_base = 32
for _k in range(1, F + 1):
    _SLAB_BASE[_k] = _base
    _g = NF - _k
    _base += -(-_g // 8) * 8
K_PAD = _base

_ROWSRC = np.zeros(K_PAD, np.int32)
_ROWVALID = np.zeros(K_PAD, np.float32)
_ROWSRC[0:D] = np.arange(D)
_ROWVALID[0:D] = 1.0
for _k in range(1, F + 1):
    for _g in range(NF - _k):
        _pos = _SLAB_BASE[_k] + _g
        _ROWSRC[_pos] = D + _PAIRPOS[(_g + _k, _g)]
        _ROWVALID[_pos] = 1.0

BT = 256


def _tc_body(xT_ref, emb_ref, dW1T_ref, db1_ref, dW2T_ref, db2_ref,
             dW3T_ref, db3_ref, oW1pT_ref, ob1_ref, oW2T_ref, ob2_ref,
             oW3T_ref, ob3_ref, oW4T_ref, ob4_ref, out_ref,
             comb_ref, rT_ref):
    j = pl.program_id(0)

    @pl.when(j == 0)
    def _init():
        rT_ref[...] = jnp.zeros((K_PAD, BT), jnp.float32)

    x = xT_ref[...]
    h1 = jnp.maximum(jnp.dot(dW1T_ref[...], x,
                             preferred_element_type=jnp.float32)
                     + db1_ref[...], 0.0)
    h2 = jnp.maximum(jnp.dot(dW2T_ref[...], h1,
                             preferred_element_type=jnp.float32)
                     + db2_ref[...], 0.0)
    h = jnp.maximum(jnp.dot(dW3T_ref[...], h2,
                            preferred_element_type=jnp.float32)
                    + db3_ref[...], 0.0)

    embT = jnp.transpose(emb_ref[...])
    comb_ref[0] = h
    comb_ref[pl.ds(1, F)] = embT.reshape(F, D, BT)

    comb = comb_ref[...]
    rT_ref[pl.ds(0, D)] = h
    for k in range(1, F + 1):
        g = NF - k
        s = jnp.sum(comb[k:NF] * comb[0:g], axis=1)
        rT_ref[pl.ds(_SLAB_BASE[k], g)] = s

    r = rT_ref[...]
    o1 = jnp.maximum(jnp.dot(oW1pT_ref[...], r,
                             preferred_element_type=jnp.float32)
                     + ob1_ref[...], 0.0)
    o2 = jnp.maximum(jnp.dot(oW2T_ref[...], o1,
                             preferred_element_type=jnp.float32)
                     + ob2_ref[...], 0.0)
    o3 = jnp.maximum(jnp.dot(oW3T_ref[...], o2,
                             preferred_element_type=jnp.float32)
                     + ob3_ref[...], 0.0)
    out_ref[...] = jnp.dot(oW4T_ref[...], o3,
                           preferred_element_type=jnp.float32) + ob4_ref[...]


def _full(shape):
    return pl.BlockSpec(shape, lambda j: (0,) * len(shape))


_tc_call = pl.pallas_call(
    _tc_body,
    grid=(B // BT,),
    in_specs=[
        pl.BlockSpec((DIN, BT), lambda j: (0, j)),
        pl.BlockSpec((BT, F * D), lambda j: (j, 0)),
        _full((512, DIN)), _full((512, 1)),
        _full((256, 512)), _full((256, 1)),
        _full((D, 256)), _full((D, 1)),
        _full((512, K_PAD)), _full((512, 1)),
        _full((512, 512)), _full((512, 1)),
        _full((256, 512)), _full((256, 1)),
        _full((1, 256)), _full((1, 1)),
    ],
    out_specs=pl.BlockSpec((1, BT), lambda j: (0, j)),
    out_shape=jax.ShapeDtypeStruct((1, B), jnp.float32),
    scratch_shapes=[
        pltpu.VMEM((NF, D, BT), jnp.float32),
        pltpu.VMEM((K_PAD, BT), jnp.float32),
    ],
)


def kernel(dense_features, indices, table, dW1, db1, dW2, db2, dW3, db3,
           oW1, ob1, oW2, ob2, oW3, ob3, oW4, ob4):
    idx2 = indices.reshape(B * F // NIDX, NIDX)
    emb_flat = _sc_gather(idx2, jnp.asarray(_OFF2D), table)
    emb2 = emb_flat.reshape(B, F * D)

    oW1p = oW1[_ROWSRC] * _ROWVALID[:, None]
    logitsT = _tc_call(
        dense_features.T, emb2,
        dW1.T, db1.reshape(512, 1),
        dW2.T, db2.reshape(256, 1),
        dW3.T, db3.reshape(D, 1),
        oW1p.T, ob1.reshape(512, 1),
        oW2.T, ob2.reshape(512, 1),
        oW3.T, ob3.reshape(256, 1),
        oW4.T, ob4.reshape(1, 1),
    )
    return logitsT.reshape(B, 1)


# speedup vs baseline: 2.1396x; 2.1396x over previous
import functools

import numpy as np
import jax
import jax.numpy as jnp
from jax import lax
from jax.experimental import pallas as pl
from jax.experimental.pallas import tpu as pltpu
from jax.experimental.pallas import tpu_sc as plsc

B = 4096
F = 26
V = 100000
D = 32
DIN = 13
NF = F + 1

NC = 2
NS = 16
NW = NC * NS
CHUNK = B * F // NW
NIDX = 128
NCH = CHUNK // NIDX

_OFF = ((np.arange(CHUNK, dtype=np.int64) % F) * V).astype(np.int32)
_OFF2D = _OFF.reshape(NCH, NIDX)

@functools.cache
def _sc_gather_fn():
    mesh = plsc.VectorSubcoreMesh(core_axis_name="c", subcore_axis_name="s")

    @functools.partial(
        pl.kernel,
        mesh=mesh,
        out_type=jax.ShapeDtypeStruct((B * F, D), jnp.float32),
        compiler_params=pltpu.CompilerParams(use_tc_tiling_on_sc=False),
        scratch_types=[
            pltpu.VMEM((NCH, NIDX), jnp.int32),
            pltpu.VMEM((NCH, NIDX), jnp.int32),
            pltpu.VMEM((CHUNK, D), jnp.float32),
            pltpu.SemaphoreType.DMA,
        ],
    )
    def _sc_gather(idx_hbm, off_hbm, table_hbm, out_hbm, idx_v, off_v, rows_v, sem):
        wid = lax.axis_index("s") * NC + lax.axis_index("c")
        pltpu.sync_copy(idx_hbm.at[wid], idx_v)
        pltpu.sync_copy(off_hbm, off_v)
        for j in range(NCH):
            for i in range(NIDX // 16):
                sl = pl.ds(i * 16, 16)
                idx_v[j, sl] = idx_v[j, sl] + off_v[j, sl]
        cps = []
        for j in range(NCH):
            cps.append(
                pltpu.async_copy(
                    table_hbm.at[idx_v.at[j]],
                    rows_v.at[pl.ds(j * NIDX, NIDX)],
                    sem,
                )
            )
        for cp in cps:
            cp.wait()
        pltpu.sync_copy(rows_v, out_hbm.at[pl.ds(wid * CHUNK, CHUNK)])

    return _sc_gather


_TRI_I, _TRI_J = np.tril_indices(NF, -1)
_PAIRPOS = {(int(i), int(j)): c for c, (i, j) in enumerate(zip(_TRI_I, _TRI_J))}

_SLAB_BASE = [0] * (F + 1)
_base = 32
for _k in range(1, F + 1):
    _SLAB_BASE[_k] = _base
    _g = NF - _k
    _base += -(-_g // 8) * 8
K_PAD = _base

_ROWSRC = np.zeros(K_PAD, np.int32)
_ROWVALID = np.zeros(K_PAD, np.float32)
_ROWSRC[0:D] = np.arange(D)
_ROWVALID[0:D] = 1.0
for _k in range(1, F + 1):
    for _g in range(NF - _k):
        _pos = _SLAB_BASE[_k] + _g
        _ROWSRC[_pos] = D + _PAIRPOS[(_g + _k, _g)]
        _ROWVALID[_pos] = 1.0

BT = 256


def _tc_body(xT_ref, emb_ref, dW1T_ref, db1_ref, dW2T_ref, db2_ref,
             dW3T_ref, db3_ref, oW1pT_ref, ob1_ref, oW2T_ref, ob2_ref,
             oW3T_ref, ob3_ref, oW4T_ref, ob4_ref, out_ref,
             comb_ref, rT_ref):
    j = pl.program_id(0)

    @pl.when(j == 0)
    def _init():
        rT_ref[...] = jnp.zeros((K_PAD, BT), jnp.float32)

    x = xT_ref[...]
    h1 = jnp.maximum(jnp.dot(dW1T_ref[...], x,
                             preferred_element_type=jnp.float32)
                     + db1_ref[...], 0.0)
    h2 = jnp.maximum(jnp.dot(dW2T_ref[...], h1,
                             preferred_element_type=jnp.float32)
                     + db2_ref[...], 0.0)
    h = jnp.maximum(jnp.dot(dW3T_ref[...], h2,
                            preferred_element_type=jnp.float32)
                    + db3_ref[...], 0.0)

    embT = jnp.transpose(emb_ref[...])
    comb_ref[0] = h
    comb_ref[pl.ds(1, F)] = embT.reshape(F, D, BT)

    comb = comb_ref[...]
    rT_ref[pl.ds(0, D)] = h
    for k in range(1, F + 1):
        g = NF - k
        s = jnp.sum(comb[k:NF] * comb[0:g], axis=1)
        rT_ref[pl.ds(_SLAB_BASE[k], g)] = s

    r = rT_ref[...]
    o1 = jnp.maximum(jnp.dot(oW1pT_ref[...], r,
                             preferred_element_type=jnp.float32)
                     + ob1_ref[...], 0.0)
    o2 = jnp.maximum(jnp.dot(oW2T_ref[...], o1,
                             preferred_element_type=jnp.float32)
                     + ob2_ref[...], 0.0)
    o3 = jnp.maximum(jnp.dot(oW3T_ref[...], o2,
                             preferred_element_type=jnp.float32)
                     + ob3_ref[...], 0.0)
    out_ref[...] = jnp.dot(oW4T_ref[...], o3,
                           preferred_element_type=jnp.float32) + ob4_ref[...]


def _full(shape):
    return pl.BlockSpec(shape, lambda j: (0,) * len(shape))


_tc_call = pl.pallas_call(
    _tc_body,
    grid=(B // BT,),
    in_specs=[
        pl.BlockSpec((DIN, BT), lambda j: (0, j)),
        pl.BlockSpec((BT, F * D), lambda j: (j, 0)),
        _full((512, DIN)), _full((512, 1)),
        _full((256, 512)), _full((256, 1)),
        _full((D, 256)), _full((D, 1)),
        _full((512, K_PAD)), _full((512, 1)),
        _full((512, 512)), _full((512, 1)),
        _full((256, 512)), _full((256, 1)),
        _full((1, 256)), _full((1, 1)),
    ],
    out_specs=pl.BlockSpec((1, BT), lambda j: (0, j)),
    out_shape=jax.ShapeDtypeStruct((1, B), jnp.float32),
    scratch_shapes=[
        pltpu.VMEM((NF, D, BT), jnp.float32),
        pltpu.VMEM((K_PAD, BT), jnp.float32),
    ],
)


def kernel(dense_features, indices, table, dW1, db1, dW2, db2, dW3, db3,
           oW1, ob1, oW2, ob2, oW3, ob3, oW4, ob4):
    idx2 = indices.reshape(NW, NCH, NIDX)
    emb_flat = _sc_gather_fn()(idx2, jnp.asarray(_OFF2D), table)
    emb2 = emb_flat.reshape(B, F * D)

    oW1p = oW1[_ROWSRC] * _ROWVALID[:, None]
    logitsT = _tc_call(
        dense_features.T, emb2,
        dW1.T, db1.reshape(512, 1),
        dW2.T, db2.reshape(256, 1),
        dW3.T, db3.reshape(D, 1),
        oW1p.T, ob1.reshape(512, 1),
        oW2.T, ob2.reshape(512, 1),
        oW3.T, ob3.reshape(256, 1),
        oW4.T, ob4.reshape(1, 1),
    )
    return logitsT.reshape(B, 1)


# speedup vs baseline: 9.7046x; 4.5356x over previous
import functools

import numpy as np
import jax
import jax.numpy as jnp
from jax import lax
from jax.experimental import pallas as pl
from jax.experimental.pallas import tpu as pltpu
from jax.experimental.pallas import tpu_sc as plsc

B = 4096
F = 26
V = 100000
D = 32
DIN = 13
NF = F + 1

NC = 2
NS = 16
NW = NC * NS
CHUNK = B * F // NW
NIDX = 128
NCH = CHUNK // NIDX

_OFF = ((np.arange(CHUNK, dtype=np.int64) % F) * V).astype(np.int32)
_OFF2D = _OFF.reshape(NCH, NIDX)

@functools.cache
def _sc_gather_fn():
    mesh = plsc.VectorSubcoreMesh(core_axis_name="c", subcore_axis_name="s")

    @functools.partial(
        pl.kernel,
        mesh=mesh,
        out_type=jax.ShapeDtypeStruct((B * F, D), jnp.float32),
        compiler_params=pltpu.CompilerParams(use_tc_tiling_on_sc=False),
        scratch_types=[
            pltpu.VMEM((NCH, NIDX), jnp.int32),
            pltpu.VMEM((NCH, NIDX), jnp.int32),
            pltpu.VMEM((CHUNK, D), jnp.float32),
            pltpu.SemaphoreType.DMA,
        ],
    )
    def _sc_gather(idx_hbm, off_hbm, table_hbm, out_hbm, idx_v, off_v, rows_v, sem):
        wid = lax.axis_index("s") * NC + lax.axis_index("c")
        pltpu.sync_copy(idx_hbm.at[wid], idx_v)
        pltpu.sync_copy(off_hbm, off_v)
        for j in range(NCH):
            for i in range(NIDX // 16):
                sl = pl.ds(i * 16, 16)
                fi = idx_v[j, sl] + off_v[j, sl]
                q = (3
                     - lax.shift_right_logical(fi - QROWS, 31)
                     - lax.shift_right_logical(fi - 2 * QROWS, 31)
                     - lax.shift_right_logical(fi - 3 * QROWS, 31))
                idx_v[j, sl] = (fi - q * QROWS) * 4 + q
        cps = []
        for j in range(NCH):
            cps.append(
                pltpu.async_copy(
                    table_hbm.at[idx_v.at[j]],
                    rows_v.at[pl.ds(j * NIDX, NIDX)],
                    sem,
                )
            )
        for cp in cps:
            cp.wait()
        pltpu.sync_copy(rows_v, out_hbm.at[pl.ds(wid * CHUNK, CHUNK)])

    return _sc_gather


RROWS = 20480
_RG = 32
QROWS = RROWS * _RG


def _repack_body(t0_ref, t1_ref, t2_ref, t3_ref, out_ref):
    x = jnp.concatenate(
        [t0_ref[...], t1_ref[...], t2_ref[...], t3_ref[...]], axis=0)
    eye = jnp.eye(128, dtype=jnp.float32)
    out_ref[...] = jax.lax.dot_general(
        x, eye, (((0,), (0,)), ((), ())),
        preferred_element_type=jnp.float32)


_MAXBLK = (F * V - 1) // RROWS


def _q_spec(q):
    return pl.BlockSpec(
        (D, RROWS),
        lambda j, _q=q: (0, jnp.minimum(_q * _RG + j, _MAXBLK)))


_repack_call = pl.pallas_call(
    _repack_body,
    grid=(_RG,),
    in_specs=[_q_spec(0), _q_spec(1), _q_spec(2), _q_spec(3)],
    out_specs=pl.BlockSpec((RROWS, 128), lambda j: (j, 0)),
    out_shape=jax.ShapeDtypeStruct((QROWS, 128), jnp.float32),
)


_TRI_I, _TRI_J = np.tril_indices(NF, -1)
_PAIRPOS = {(int(i), int(j)): c for c, (i, j) in enumerate(zip(_TRI_I, _TRI_J))}

_SLAB_BASE = [0] * (F + 1)
_base = 32
for _k in range(1, F + 1):
    _SLAB_BASE[_k] = _base
    _g = NF - _k
    _base += -(-_g // 8) * 8
K_PAD = _base

_ROWSRC = np.zeros(K_PAD, np.int32)
_ROWVALID = np.zeros(K_PAD, np.float32)
_ROWSRC[0:D] = np.arange(D)
_ROWVALID[0:D] = 1.0
for _k in range(1, F + 1):
    for _g in range(NF - _k):
        _pos = _SLAB_BASE[_k] + _g
        _ROWSRC[_pos] = D + _PAIRPOS[(_g + _k, _g)]
        _ROWVALID[_pos] = 1.0

BT = 512


def _tc_body(xT_ref, emb_ref, dW1T_ref, db1_ref, dW2T_ref, db2_ref,
             dW3T_ref, db3_ref, oW1pT_ref, ob1_ref, oW2T_ref, ob2_ref,
             oW3T_ref, ob3_ref, oW4T_ref, ob4_ref, out_ref,
             comb_ref, rT_ref):
    j = pl.program_id(0)

    @pl.when(j == 0)
    def _init():
        rT_ref[...] = jnp.zeros((K_PAD, BT), jnp.float32)

    x = xT_ref[...]
    h1 = jnp.maximum(jnp.dot(dW1T_ref[...], x,
                             preferred_element_type=jnp.float32)
                     + db1_ref[...], 0.0)
    h2 = jnp.maximum(jnp.dot(dW2T_ref[...], h1,
                             preferred_element_type=jnp.float32)
                     + db2_ref[...], 0.0)
    h = jnp.maximum(jnp.dot(dW3T_ref[...], h2,
                            preferred_element_type=jnp.float32)
                    + db3_ref[...], 0.0)

    embT = jnp.transpose(emb_ref[...])
    comb_ref[0] = h
    comb_ref[pl.ds(1, F)] = embT.reshape(F, D, BT)

    comb = comb_ref[...]
    rT_ref[pl.ds(0, D)] = h
    for k in range(1, F + 1):
        g = NF - k
        s = jnp.sum(comb[k:NF] * comb[0:g], axis=1)
        rT_ref[pl.ds(_SLAB_BASE[k], g)] = s

    r = rT_ref[...]
    o1 = jnp.maximum(jnp.dot(oW1pT_ref[...], r,
                             preferred_element_type=jnp.float32)
                     + ob1_ref[...], 0.0)
    o2 = jnp.maximum(jnp.dot(oW2T_ref[...], o1,
                             preferred_element_type=jnp.float32)
                     + ob2_ref[...], 0.0)
    o3 = jnp.maximum(jnp.dot(oW3T_ref[...], o2,
                             preferred_element_type=jnp.float32)
                     + ob3_ref[...], 0.0)
    out_ref[...] = jnp.dot(oW4T_ref[...], o3,
                           preferred_element_type=jnp.float32) + ob4_ref[...]


def _full(shape):
    return pl.BlockSpec(shape, lambda j: (0,) * len(shape))


_tc_call = pl.pallas_call(
    _tc_body,
    grid=(B // BT,),
    in_specs=[
        pl.BlockSpec((DIN, BT), lambda j: (0, j)),
        pl.BlockSpec((BT, F * D), lambda j: (j, 0)),
        _full((512, DIN)), _full((512, 1)),
        _full((256, 512)), _full((256, 1)),
        _full((D, 256)), _full((D, 1)),
        _full((512, K_PAD)), _full((512, 1)),
        _full((512, 512)), _full((512, 1)),
        _full((256, 512)), _full((256, 1)),
        _full((1, 256)), _full((1, 1)),
    ],
    out_specs=pl.BlockSpec((1, BT), lambda j: (0, j)),
    out_shape=jax.ShapeDtypeStruct((1, B), jnp.float32),
    scratch_shapes=[
        pltpu.VMEM((NF, D, BT), jnp.float32),
        pltpu.VMEM((K_PAD, BT), jnp.float32),
    ],
)


def kernel(dense_features, indices, table, dW1, db1, dW2, db2, dW3, db3,
           oW1, ob1, oW2, ob2, oW3, ob3, oW4, ob4):
    idx2 = indices.reshape(NW, NCH, NIDX)
    tT = table.T
    tableC = _repack_call(tT, tT, tT, tT).reshape(4 * QROWS, D)
    emb_flat = _sc_gather_fn()(idx2, jnp.asarray(_OFF2D), tableC)
    emb2 = emb_flat.reshape(B, F * D)

    oW1p = oW1[_ROWSRC] * _ROWVALID[:, None]
    logitsT = _tc_call(
        dense_features.T, emb2,
        dW1.T, db1.reshape(512, 1),
        dW2.T, db2.reshape(256, 1),
        dW3.T, db3.reshape(D, 1),
        oW1p.T, ob1.reshape(512, 1),
        oW2.T, ob2.reshape(512, 1),
        oW3.T, ob3.reshape(256, 1),
        oW4.T, ob4.reshape(1, 1),
    )
    return logitsT.reshape(B, 1)
